# Initial kernel scaffold; baseline (speedup 1.0000x reference)
import jax
import jax.numpy as jnp
from jax.experimental import pallas as pl


def kernel(x, b_pre, W_enc, b_enc, W_dec, b_dec):
    raise NotImplementedError("write your pallas kernel here")


# speedup vs baseline: 12.8618x; 12.8618x over previous
import jax
import jax.numpy as jnp
from jax.experimental import pallas as pl

K = 32
BLOCK_ROWS = 256


def _sae_block(x_ref, b_pre_ref, w_enc_ref, b_enc_ref, w_dec_ref, b_dec_ref,
               recon_ref, sparse_ref):
    x_c = x_ref[...] - b_pre_ref[...]
    latents = jax.lax.dot_general(
        x_c, w_enc_ref[...], (((1,), (1,)), ((), ())),
        preferred_element_type=jnp.float32,
    ) + b_enc_ref[...]

    def body(i, cur):
        m = jnp.max(cur, axis=1, keepdims=True)
        return jnp.where(cur == m, -jnp.inf, cur)

    cur = jax.lax.fori_loop(0, K - 1, body, latents)
    thresh = jnp.max(cur, axis=1, keepdims=True)

    sparse = jnp.where(latents >= thresh, latents, 0.0)
    sparse_ref[...] = sparse

    recon_ref[...] = jax.lax.dot_general(
        sparse, w_dec_ref[...], (((1,), (1,)), ((), ())),
        preferred_element_type=jnp.float32,
    ) + b_dec_ref[...] + b_pre_ref[...]


def kernel(x, b_pre, W_enc, b_enc, W_dec, b_dec):
    n, d = x.shape
    latent_dim = W_enc.shape[0]
    grid = (n // BLOCK_ROWS,)
    b_pre2 = b_pre.reshape(1, d)
    b_enc2 = b_enc.reshape(1, latent_dim)
    b_dec2 = b_dec.reshape(1, d)

    recon, sparse = pl.pallas_call(
        _sae_block,
        grid=grid,
        in_specs=[
            pl.BlockSpec((BLOCK_ROWS, d), lambda i: (i, 0)),
            pl.BlockSpec((1, d), lambda i: (0, 0)),
            pl.BlockSpec((latent_dim, d), lambda i: (0, 0)),
            pl.BlockSpec((1, latent_dim), lambda i: (0, 0)),
            pl.BlockSpec((d, latent_dim), lambda i: (0, 0)),
            pl.BlockSpec((1, d), lambda i: (0, 0)),
        ],
        out_specs=[
            pl.BlockSpec((BLOCK_ROWS, d), lambda i: (i, 0)),
            pl.BlockSpec((BLOCK_ROWS, latent_dim), lambda i: (i, 0)),
        ],
        out_shape=[
            jax.ShapeDtypeStruct((n, d), jnp.float32),
            jax.ShapeDtypeStruct((n, latent_dim), jnp.float32),
        ],
    )(x, b_pre2, W_enc, b_enc2, W_dec, b_dec2)
    return (recon, sparse)
